# baseline (speedup 1.0000x reference)
import functools

import jax
import jax.numpy as jnp
from jax import lax
from jax.experimental import pallas as pl
from jax.experimental.pallas import tpu as pltpu
from jax.experimental.pallas import tpu_sc as plsc

NCELL = 20000
NGENE = 2000
NCP = 20480
NGP = 2048
D = 128
DH = 64
E = 320000
ER = E // 128
ERP = 2560
RPT = ERP // 16
DRP = 5120
THIRD = 1.0 / 3.0

_MESH = plsc.VectorSubcoreMesh(core_axis_name="c", subcore_axis_name="s")

def _z16():
    return jnp.zeros((16,), jnp.float32)


def _ones16():
    return jnp.ones((16,), jnp.float32)


@functools.partial(
    pl.kernel,
    out_type=[
        jax.ShapeDtypeStruct((16, NCP), jnp.float32),
        jax.ShapeDtypeStruct((16, NGP), jnp.float32),
    ],
    mesh=_MESH,
    compiler_params=pltpu.CompilerParams(needs_layout_passes=False, use_tc_tiling_on_sc=False),
    scratch_types=[
        pltpu.VMEM((NCP,), jnp.float32),
        pltpu.VMEM((RPT, 128), jnp.int32),
    ],
)
def _deg_hist(src2d, dst2d, hc, hg, hist, iball):
    c = lax.axis_index("c")
    s = lax.axis_index("s")
    rb = pl.multiple_of(s * RPT, 8)

    @pl.when(c == 0)
    def _():
        pltpu.sync_copy(src2d.at[pl.ds(rb, RPT)], iball)

    @pl.when(c == 1)
    def _():
        pltpu.sync_copy(dst2d.at[pl.ds(rb, RPT)], iball)

    def zbody(i, _):
        hist[pl.ds(pl.multiple_of(i * 16, 16), 16)] = _z16()
        return 0

    lax.fori_loop(0, NCP // 16, zbody, 0)

    def ebody(i, _):
        for l in range(8):
            iv = iball[i, pl.ds(l * 16, 16)]
            plsc.addupdate_scatter(hist, [iv], _ones16())
        return 0

    lax.fori_loop(0, RPT, ebody, 0)

    @pl.when(c == 0)
    def _():
        pltpu.sync_copy(hist, hc.at[s])

    @pl.when(c == 1)
    def _():
        pltpu.sync_copy(hist.at[pl.ds(0, NGP)], hg.at[s])


def _reduce_cc_body(h_ref, cc_ref):
    deg = jnp.sum(h_ref[...], axis=0)
    cc_ref[...] = jnp.where(deg > 0, lax.rsqrt(deg), 0.0)


def _make_reduce_cc(n, bc):
    return pl.pallas_call(
        _reduce_cc_body,
        grid=(n // bc,),
        in_specs=[pl.BlockSpec((16, bc), lambda i: (0, i))],
        out_specs=pl.BlockSpec((bc,), lambda i: (i,)),
        out_shape=jax.ShapeDtypeStruct((n,), jnp.float32),
    )


_reduce_cc_cells = _make_reduce_cc(NCP, 1024)
_reduce_cc_genes = _make_reduce_cc(NGP, 1024)


RPT2 = 320


@functools.partial(
    pl.kernel,
    out_type=[
        jax.ShapeDtypeStruct((2, NCP, DH), jnp.float32),
        jax.ShapeDtypeStruct((2, NGP, DH), jnp.float32),
    ],
    mesh=_MESH,
    compiler_params=pltpu.CompilerParams(needs_layout_passes=False, use_tc_tiling_on_sc=False),
    name="gcn_layer",
    scratch_types=[
        pltpu.VMEM_SHARED((NCP, DH), jnp.float32),
        pltpu.VMEM_SHARED((NGP, DH), jnp.float32),
        [pltpu.VMEM((1, 64), jnp.int32)] * 8,
        [pltpu.VMEM((1, 64), jnp.int32)] * 8,
        [pltpu.VMEM((64, DH), jnp.float32)] * 4,
        [pltpu.VMEM((64, DH), jnp.float32)] * 4,
        [pltpu.SemaphoreType.DMA] * 24,
    ],
)
def _layer(ah, bh, src2d, dst2d, csum, gsum, sca, sga, ibs, ibd, rba, rbb, sems):
    c = lax.axis_index("c")
    s = lax.axis_index("s")
    sga_g = sems[0:4]
    sgb_g = sems[4:8]
    ssa_g = sems[8:12]
    ssb_g = sems[12:16]
    sem_i = sems[16:24]

    def zbody(i, _):
        for q in range(4):
            rba[0][i, pl.ds(q * 16, 16)] = _z16()
        return 0

    lax.fori_loop(0, 64, zbody, 0)
    cbase = pl.multiple_of(s * 1280, 8)
    gbase = pl.multiple_of(s * 128, 8)
    rb = pl.multiple_of(s * RPT2, 8)
    for k in range(20):
        pltpu.sync_copy(rba[0], sca.at[pl.ds(cbase + k * 64, 64)])
    for k in range(2):
        pltpu.sync_copy(rba[0], sga.at[pl.ds(gbase + k * 64, 64)])
    plsc.subcore_barrier()

    def fire_idx(g, s8):
        pltpu.async_copy(src2d.at[rb + g], ibs[s8].at[0], sem_i[s8])
        pltpu.async_copy(dst2d.at[rb + g], ibd[s8].at[0], sem_i[s8])

    def drain_idx(s8):
        pltpu.make_async_copy(src2d.at[rb], ibs[s8].at[0], sem_i[s8]).wait()
        pltpu.make_async_copy(src2d.at[rb], ibd[s8].at[0], sem_i[s8]).wait()

    def fire_gathers(s4, s8):
        pltpu.async_copy(ah.at[c].at[ibs[s8].at[0]], rba[s4], sga_g[s4])
        pltpu.async_copy(bh.at[c].at[ibd[s8].at[0]], rbb[s4], sgb_g[s4])

    def drain_gathers(s4):
        pltpu.make_async_copy(ah.at[c].at[ibs[0].at[0]], rba[s4], sga_g[s4]).wait()
        pltpu.make_async_copy(bh.at[c].at[ibd[0].at[0]], rbb[s4], sgb_g[s4]).wait()

    def fire_scatters(s4, s8):
        pltpu.async_copy(rba[s4], sga.at[ibd[s8].at[0]], ssa_g[s4], add=True)
        pltpu.async_copy(rbb[s4], sca.at[ibs[s8].at[0]], ssb_g[s4], add=True)

    def drain_scatters(s4):
        pltpu.make_async_copy(rba[s4], sga.at[ibd[0].at[0]], ssa_g[s4]).wait()
        pltpu.make_async_copy(rbb[s4], sca.at[ibs[0].at[0]], ssb_g[s4]).wait()

    for t in range(4):
        fire_idx(t, t)
    for t in range(3):
        drain_idx(t)
        fire_gathers(t, t)

    def gstep(g, t, fire_i=True, fire_g=True, first=False):
        s4 = t % 4
        drain_gathers(s4)
        if not first:
            drain_scatters((t + 3) % 4)
        if fire_i:
            fire_idx(g + 4, (t + 4) % 8)
        if fire_g:
            drain_idx((t + 3) % 8)
            fire_gathers((t + 3) % 4, (t + 3) % 8)
        fire_scatters(s4, t)

    NM2 = RPT2 // 8

    def mbody(m, _):
        g0 = m * 8

        @pl.when(m == 0)
        def _():
            gstep(g0, 0, first=True)

        @pl.when(m > 0)
        def _():
            gstep(g0, 0)

        for t in range(1, 4):
            gstep(g0 + t, t)

        @pl.when(m < NM2 - 1)
        def _():
            for t in range(4, 8):
                gstep(g0 + t, t)

        @pl.when(m == NM2 - 1)
        def _():
            gstep(g0 + 4, 4, fire_i=False)
            for t in range(5, 8):
                gstep(g0 + t, t, fire_i=False, fire_g=False)

        return 0

    lax.fori_loop(0, NM2, mbody, 0)
    drain_scatters(3)
    plsc.subcore_barrier()

    for k in range(2):
        o = pl.ds(cbase + k * 640, 640)
        pltpu.sync_copy(sca.at[o], csum.at[c].at[o])
    pltpu.sync_copy(sga.at[pl.ds(gbase, 128)], gsum.at[c].at[pl.ds(gbase, 128)])


_DEPTH = 16


@functools.partial(
    pl.kernel,
    out_type=[jax.ShapeDtypeStruct((DRP, 128), jnp.float32)],
    mesh=_MESH,
    compiler_params=pltpu.CompilerParams(needs_layout_passes=False, use_tc_tiling_on_sc=False),
    scratch_types=[
        pltpu.VMEM((160, 128), jnp.int32),
        pltpu.VMEM((160, 128), jnp.int32),
        pltpu.VMEM((160, 128), jnp.int32),
        pltpu.VMEM((160, 128), jnp.float32),
        pltpu.SemaphoreType.DMA,
    ],
)
def _decode(gf, csrc, cdst, pred, sidx, didx, ostage, ostage2, sem):
    c = lax.axis_index("c")
    s = lax.axis_index("s")
    wid = s * 2 + c
    rb = pl.multiple_of(wid * 160, 8)
    pltpu.sync_copy(csrc.at[pl.ds(rb, 160)], sidx)
    pltpu.sync_copy(cdst.at[pl.ds(rb, 160)], didx)

    def fbody(i, _):
        for l in range(8):
            o = pl.ds(l * 16, 16)
            sidx[i, o] = sidx[i, o] * (NGP // 2) + (didx[i, o] >> 1)
        return 0

    lax.fori_loop(0, 160, fbody, 0)

    def rbody(i, _):
        pltpu.async_copy(gf.at[sidx.at[i]], ostage.at[i], sem)

        @pl.when(i >= _DEPTH)
        def _():
            pltpu.make_async_copy(gf.at[sidx.at[0]], ostage.at[i - _DEPTH], sem).wait()

        return 0

    lax.fori_loop(0, 160, rbody, 0)

    def dbody(i, _):
        pltpu.make_async_copy(gf.at[sidx.at[0]], ostage.at[160 - _DEPTH + i], sem).wait()
        return 0

    lax.fori_loop(0, _DEPTH, dbody, 0)

    def xbody(i, _):
        for l in range(8):
            o = pl.ds(l * 16, 16)
            w = ostage[i, o]
            p = didx[i, o] & 1
            bits = jnp.where(p == 1, w & jnp.int32(-65536), w << 16)
            ostage2[i, o] = plsc.bitcast(bits, jnp.float32)
        return 0

    lax.fori_loop(0, 160, xbody, 0)
    pltpu.sync_copy(ostage2, pred.at[pl.ds(rb, 160)])


def _prescale_body(x_ref, cc_ref, a_ref, h_ref):
    sc = cc_ref[...]
    for j in range(2):
        x = x_ref[:, j * DH:(j + 1) * DH]
        a_ref[j] = x * sc
        h_ref[j] = x * THIRD


def _make_prescale(n, br):
    grid = (n // br,)
    return pl.pallas_call(
        _prescale_body,
        grid=grid,
        in_specs=[
            pl.BlockSpec((br, D), lambda i: (i, 0)),
            pl.BlockSpec((br, 1), lambda i: (i, 0)),
        ],
        out_specs=[
            pl.BlockSpec((2, br, DH), lambda i: (0, i, 0)),
            pl.BlockSpec((2, br, DH), lambda i: (0, i, 0)),
        ],
        out_shape=[
            jax.ShapeDtypeStruct((2, n, DH), jnp.float32),
            jax.ShapeDtypeStruct((2, n, DH), jnp.float32),
        ],
    )


def _update_body(sum_ref, cc_ref, hprev_ref, hnew_ref, anext_ref):
    sc = cc_ref[...]
    t = sum_ref[0] * sc
    hnew_ref[0] = hprev_ref[0] + THIRD * t
    anext_ref[0] = t * sc


def _make_update(n, br):
    grid = (n // br, 2)
    return pl.pallas_call(
        _update_body,
        grid=grid,
        in_specs=[
            pl.BlockSpec((1, br, DH), lambda i, j: (j, i, 0)),
            pl.BlockSpec((br, 1), lambda i, j: (i, 0)),
            pl.BlockSpec((1, br, DH), lambda i, j: (j, i, 0)),
        ],
        out_specs=[
            pl.BlockSpec((1, br, DH), lambda i, j: (j, i, 0)),
            pl.BlockSpec((1, br, DH), lambda i, j: (j, i, 0)),
        ],
        out_shape=[
            jax.ShapeDtypeStruct((2, n, DH), jnp.float32),
            jax.ShapeDtypeStruct((2, n, DH), jnp.float32),
        ],
    )


_prescale_cells = _make_prescale(NCP, 1024)
_update_cells = _make_update(NCP, 1024)

_prescale_genes = _make_prescale(NGP, 512)
_update_genes = _make_update(NGP, 512)


_DN = (((1,), (1,)), ((), ()))


def _gmm_body(u_ref, v_ref, o_ref):
    o_ref[...] = (
        lax.dot_general(u_ref[0], v_ref[0], _DN, preferred_element_type=jnp.float32)
        + lax.dot_general(u_ref[1], v_ref[1], _DN, preferred_element_type=jnp.float32)
    ).astype(jnp.bfloat16)


_gmm = pl.pallas_call(
    _gmm_body,
    grid=(NCP // 512, NGP // 512),
    in_specs=[
        pl.BlockSpec((2, 512, DH), lambda i, j: (0, i, 0)),
        pl.BlockSpec((2, 512, DH), lambda i, j: (0, j, 0)),
    ],
    out_specs=pl.BlockSpec((512, 512), lambda i, j: (i, j)),
    out_shape=jax.ShapeDtypeStruct((NCP, NGP), jnp.bfloat16),
)


def kernel(enc_src, enc_dst, pos_src, pos_dst, neg_src, neg_dst, cell_feat, gene_feat):
    npad = ERP * 128 - E
    es = jnp.concatenate(
        [enc_src.astype(jnp.int32),
         jnp.full((npad,), NCP - 1, jnp.int32)]).reshape(ERP, 128)
    ed = jnp.concatenate(
        [enc_dst.astype(jnp.int32),
         jnp.full((npad,), NGP - 1, jnp.int32)]).reshape(ERP, 128)
    hc, hg = _deg_hist(es, ed)
    es64 = es.reshape(2 * ERP, 64)
    ed64 = ed.reshape(2 * ERP, 64)
    cc_c = _reduce_cc_cells(hc)
    cc_g = _reduce_cc_genes(hg)
    cc_c2 = cc_c.reshape(NCP, 1)
    cc_g2 = cc_g.reshape(NGP, 1)
    u0p = jnp.pad(cell_feat, ((0, NCP - NCELL), (0, 0)))
    g0p = jnp.pad(gene_feat, ((0, NGP - NGENE), (0, 0)))
    a, uh = _prescale_cells(u0p, cc_c2)
    b, ih = _prescale_genes(g0p, cc_g2)
    for _ in range(2):
        csum, gsum = _layer(a, b, es64, ed64)
        uh, a = _update_cells(csum, cc_c2, uh)
        ih, b = _update_genes(gsum, cc_g2, ih)
    gmat = _gmm(uh, ih)
    gf = lax.bitcast_convert_type(
        gmat.reshape(NCP * NGP // 2, 2), jnp.int32)
    dpad = DRP * 128 - 2 * E
    zpad = jnp.zeros((dpad,), jnp.int32)
    csrc = jnp.concatenate(
        [pos_src.astype(jnp.int32), neg_src.astype(jnp.int32), zpad]).reshape(DRP, 128)
    cdst = jnp.concatenate(
        [pos_dst.astype(jnp.int32), neg_dst.astype(jnp.int32), zpad]).reshape(DRP, 128)
    (pred2d,) = _decode(gf, csrc, cdst)
    pred = pred2d.reshape(DRP * 128)
    u_hidden = jnp.concatenate([uh[0], uh[1]], axis=1)[:NCELL]
    i_hidden = jnp.concatenate([ih[0], ih[1]], axis=1)[:NGENE]
    return (pred[:E], pred[E:2 * E], u_hidden, i_hidden)


# speedup vs baseline: 10.8418x; 10.8418x over previous
import functools

import jax
import jax.numpy as jnp
from jax import lax
from jax.experimental import pallas as pl
from jax.experimental.pallas import tpu as pltpu
from jax.experimental.pallas import tpu_sc as plsc

NCELL = 20000
NGENE = 2000
NCP = 20480
NGP = 2048
D = 128
DH = 64
E = 320000
ER = E // 128
ERP = 2560
RPT = ERP // 16
DRP = 5120
THIRD = 1.0 / 3.0

_MESH = plsc.VectorSubcoreMesh(core_axis_name="c", subcore_axis_name="s")

def _z16():
    return jnp.zeros((16,), jnp.float32)


def _ones16():
    return jnp.ones((16,), jnp.float32)


@functools.partial(
    pl.kernel,
    out_type=[
        jax.ShapeDtypeStruct((16, NCP), jnp.float32),
        jax.ShapeDtypeStruct((16, NGP), jnp.float32),
    ],
    mesh=_MESH,
    compiler_params=pltpu.CompilerParams(needs_layout_passes=False, use_tc_tiling_on_sc=False),
    scratch_types=[
        pltpu.VMEM((NCP,), jnp.float32),
        pltpu.VMEM((RPT, 128), jnp.int32),
    ],
)
def _deg_hist(src2d, dst2d, hc, hg, hist, iball):
    c = lax.axis_index("c")
    s = lax.axis_index("s")
    rb = pl.multiple_of(s * RPT, 8)

    @pl.when(c == 0)
    def _():
        pltpu.sync_copy(src2d.at[pl.ds(rb, RPT)], iball)

    @pl.when(c == 1)
    def _():
        pltpu.sync_copy(dst2d.at[pl.ds(rb, RPT)], iball)

    def zbody(i, _):
        hist[pl.ds(pl.multiple_of(i * 16, 16), 16)] = _z16()
        return 0

    lax.fori_loop(0, NCP // 16, zbody, 0)

    def ebody(i, _):
        for l in range(8):
            iv = iball[i, pl.ds(l * 16, 16)]
            plsc.addupdate_scatter(hist, [iv], _ones16())
        return 0

    lax.fori_loop(0, RPT, ebody, 0)

    @pl.when(c == 0)
    def _():
        pltpu.sync_copy(hist, hc.at[s])

    @pl.when(c == 1)
    def _():
        pltpu.sync_copy(hist.at[pl.ds(0, NGP)], hg.at[s])


def _reduce_cc_body(h_ref, cc_ref):
    deg = jnp.sum(h_ref[...], axis=0)
    cc_ref[...] = jnp.where(deg > 0, lax.rsqrt(deg), 0.0)


def _make_reduce_cc(n, bc):
    return pl.pallas_call(
        _reduce_cc_body,
        grid=(n // bc,),
        in_specs=[pl.BlockSpec((16, bc), lambda i: (0, i))],
        out_specs=pl.BlockSpec((bc,), lambda i: (i,)),
        out_shape=jax.ShapeDtypeStruct((n,), jnp.float32),
    )


_reduce_cc_cells = _make_reduce_cc(NCP, 1024)
_reduce_cc_genes = _make_reduce_cc(NGP, 1024)


RPT2 = 320


@functools.partial(
    pl.kernel,
    out_type=[
        jax.ShapeDtypeStruct((2, NCP, DH), jnp.float32),
        jax.ShapeDtypeStruct((2, NGP, DH), jnp.float32),
    ],
    mesh=_MESH,
    compiler_params=pltpu.CompilerParams(needs_layout_passes=False, use_tc_tiling_on_sc=False),
    name="gcn_layer",
    scratch_types=[
        pltpu.VMEM_SHARED((NCP, DH), jnp.float32),
        pltpu.VMEM_SHARED((NGP, DH), jnp.float32),
        [pltpu.VMEM((1, 64), jnp.int32)] * 8,
        [pltpu.VMEM((1, 64), jnp.int32)] * 8,
        [pltpu.VMEM((64, DH), jnp.float32)] * 4,
        [pltpu.VMEM((64, DH), jnp.float32)] * 4,
        [pltpu.SemaphoreType.DMA] * 24,
    ],
)
def _layer(ah, bh, src2d, dst2d, csum, gsum, sca, sga, ibs, ibd, rba, rbb, sems):
    c = lax.axis_index("c")
    s = lax.axis_index("s")
    sga_g = sems[0:4]
    sgb_g = sems[4:8]
    ssa_g = sems[8:12]
    ssb_g = sems[12:16]
    sem_i = sems[16:24]

    def zbody(i, _):
        for q in range(4):
            rba[0][i, pl.ds(q * 16, 16)] = _z16()
        return 0

    lax.fori_loop(0, 64, zbody, 0)
    cbase = pl.multiple_of(s * 1280, 8)
    gbase = pl.multiple_of(s * 128, 8)
    rb = pl.multiple_of(s * RPT2, 8)
    for k in range(20):
        pltpu.sync_copy(rba[0], sca.at[pl.ds(cbase + k * 64, 64)])
    for k in range(2):
        pltpu.sync_copy(rba[0], sga.at[pl.ds(gbase + k * 64, 64)])
    plsc.subcore_barrier()

    def fire_idx(g, s8):
        pltpu.async_copy(src2d.at[rb + g], ibs[s8].at[0], sem_i[s8])
        pltpu.async_copy(dst2d.at[rb + g], ibd[s8].at[0], sem_i[s8])

    def drain_idx(s8):
        pltpu.make_async_copy(src2d.at[rb], ibs[s8].at[0], sem_i[s8]).wait()
        pltpu.make_async_copy(src2d.at[rb], ibd[s8].at[0], sem_i[s8]).wait()

    def fire_gathers(s4, s8):
        pltpu.async_copy(ah.at[c].at[ibs[s8].at[0]], rba[s4], sga_g[s4])
        pltpu.async_copy(bh.at[c].at[ibd[s8].at[0]], rbb[s4], sgb_g[s4])

    def drain_gathers(s4):
        pltpu.make_async_copy(ah.at[c].at[ibs[0].at[0]], rba[s4], sga_g[s4]).wait()
        pltpu.make_async_copy(bh.at[c].at[ibd[0].at[0]], rbb[s4], sgb_g[s4]).wait()

    def fire_scatters(s4, s8):
        pltpu.async_copy(rba[s4], sga.at[ibd[s8].at[0]], ssa_g[s4], add=True)
        pltpu.async_copy(rbb[s4], sca.at[ibs[s8].at[0]], ssb_g[s4], add=True)

    def drain_scatters(s4):
        pltpu.make_async_copy(rba[s4], sga.at[ibd[0].at[0]], ssa_g[s4]).wait()
        pltpu.make_async_copy(rbb[s4], sca.at[ibs[0].at[0]], ssb_g[s4]).wait()

    for t in range(4):
        fire_idx(t, t)
    for t in range(3):
        drain_idx(t)
        fire_gathers(t, t)

    def gstep(g, t, fire_i=True, fire_g=True, first=False):
        s4 = t % 4
        drain_gathers(s4)
        if not first:
            drain_scatters((t + 3) % 4)
        if fire_i:
            fire_idx(g + 4, (t + 4) % 8)
        if fire_g:
            drain_idx((t + 3) % 8)
            fire_gathers((t + 3) % 4, (t + 3) % 8)
        fire_scatters(s4, t)

    NM2 = RPT2 // 8

    def mbody(m, _):
        g0 = m * 8

        @pl.when(m == 0)
        def _():
            gstep(g0, 0, first=True)

        @pl.when(m > 0)
        def _():
            gstep(g0, 0)

        for t in range(1, 4):
            gstep(g0 + t, t)

        @pl.when(m < NM2 - 1)
        def _():
            for t in range(4, 8):
                gstep(g0 + t, t)

        @pl.when(m == NM2 - 1)
        def _():
            gstep(g0 + 4, 4, fire_i=False)
            for t in range(5, 8):
                gstep(g0 + t, t, fire_i=False, fire_g=False)

        return 0

    lax.fori_loop(0, NM2, mbody, 0)
    drain_scatters(3)
    plsc.subcore_barrier()

    for k in range(2):
        o = pl.ds(cbase + k * 640, 640)
        pltpu.sync_copy(sca.at[o], csum.at[c].at[o])
    pltpu.sync_copy(sga.at[pl.ds(gbase, 128)], gsum.at[c].at[pl.ds(gbase, 128)])


_DEPTH = 16


@functools.partial(
    pl.kernel,
    out_type=[jax.ShapeDtypeStruct((DRP, 128), jnp.float32)],
    mesh=_MESH,
    compiler_params=pltpu.CompilerParams(needs_layout_passes=False, use_tc_tiling_on_sc=False),
    scratch_types=[
        pltpu.VMEM((160, 128), jnp.int32),
        pltpu.VMEM((160, 128), jnp.int32),
        pltpu.VMEM((160, 128), jnp.float32),
        pltpu.SemaphoreType.DMA,
    ],
)
def _decode(gf, csrc, cdst, pred, sidx, didx, ostage, sem):
    c = lax.axis_index("c")
    s = lax.axis_index("s")
    wid = s * 2 + c
    rb = pl.multiple_of(wid * 160, 8)
    pltpu.sync_copy(csrc.at[pl.ds(rb, 160)], sidx)
    pltpu.sync_copy(cdst.at[pl.ds(rb, 160)], didx)

    def fbody(i, _):
        for l in range(8):
            o = pl.ds(l * 16, 16)
            sidx[i, o] = sidx[i, o] * NGP + didx[i, o]
        return 0

    lax.fori_loop(0, 160, fbody, 0)

    def rbody(i, _):
        pltpu.async_copy(gf.at[sidx.at[i]], ostage.at[i], sem)

        @pl.when(i >= _DEPTH)
        def _():
            pltpu.make_async_copy(gf.at[sidx.at[0]], ostage.at[i - _DEPTH], sem).wait()

        return 0

    lax.fori_loop(0, 160, rbody, 0)

    def dbody(i, _):
        pltpu.make_async_copy(gf.at[sidx.at[0]], ostage.at[160 - _DEPTH + i], sem).wait()
        return 0

    lax.fori_loop(0, _DEPTH, dbody, 0)
    pltpu.sync_copy(ostage, pred.at[pl.ds(rb, 160)])


def _prescale_body(x_ref, cc_ref, a_ref, h_ref):
    sc = cc_ref[...]
    for j in range(2):
        x = x_ref[:, j * DH:(j + 1) * DH]
        a_ref[j] = x * sc
        h_ref[j] = x * THIRD


def _make_prescale(n, br):
    grid = (n // br,)
    return pl.pallas_call(
        _prescale_body,
        grid=grid,
        in_specs=[
            pl.BlockSpec((br, D), lambda i: (i, 0)),
            pl.BlockSpec((br, 1), lambda i: (i, 0)),
        ],
        out_specs=[
            pl.BlockSpec((2, br, DH), lambda i: (0, i, 0)),
            pl.BlockSpec((2, br, DH), lambda i: (0, i, 0)),
        ],
        out_shape=[
            jax.ShapeDtypeStruct((2, n, DH), jnp.float32),
            jax.ShapeDtypeStruct((2, n, DH), jnp.float32),
        ],
    )


def _update_body(sum_ref, cc_ref, hprev_ref, hnew_ref, anext_ref):
    sc = cc_ref[...]
    t = sum_ref[0] * sc
    hnew_ref[0] = hprev_ref[0] + THIRD * t
    anext_ref[0] = t * sc


def _make_update(n, br):
    grid = (n // br, 2)
    return pl.pallas_call(
        _update_body,
        grid=grid,
        in_specs=[
            pl.BlockSpec((1, br, DH), lambda i, j: (j, i, 0)),
            pl.BlockSpec((br, 1), lambda i, j: (i, 0)),
            pl.BlockSpec((1, br, DH), lambda i, j: (j, i, 0)),
        ],
        out_specs=[
            pl.BlockSpec((1, br, DH), lambda i, j: (j, i, 0)),
            pl.BlockSpec((1, br, DH), lambda i, j: (j, i, 0)),
        ],
        out_shape=[
            jax.ShapeDtypeStruct((2, n, DH), jnp.float32),
            jax.ShapeDtypeStruct((2, n, DH), jnp.float32),
        ],
    )


_prescale_cells = _make_prescale(NCP, 1024)
_update_cells = _make_update(NCP, 1024)

_prescale_genes = _make_prescale(NGP, 512)
_update_genes = _make_update(NGP, 512)


_DN = (((1,), (1,)), ((), ()))


def _gmm_body(u_ref, v_ref, o_ref):
    o_ref[...] = (
        lax.dot_general(u_ref[0], v_ref[0], _DN, preferred_element_type=jnp.float32)
        + lax.dot_general(u_ref[1], v_ref[1], _DN, preferred_element_type=jnp.float32)
    )


_gmm = pl.pallas_call(
    _gmm_body,
    grid=(NCP // 512, NGP // 512),
    in_specs=[
        pl.BlockSpec((2, 512, DH), lambda i, j: (0, i, 0)),
        pl.BlockSpec((2, 512, DH), lambda i, j: (0, j, 0)),
    ],
    out_specs=pl.BlockSpec((512, 512), lambda i, j: (i, j)),
    out_shape=jax.ShapeDtypeStruct((NCP, NGP), jnp.float32),
)


def kernel(enc_src, enc_dst, pos_src, pos_dst, neg_src, neg_dst, cell_feat, gene_feat):
    npad = ERP * 128 - E
    es = jnp.concatenate(
        [enc_src.astype(jnp.int32),
         jnp.full((npad,), NCP - 1, jnp.int32)]).reshape(ERP, 128)
    ed = jnp.concatenate(
        [enc_dst.astype(jnp.int32),
         jnp.full((npad,), NGP - 1, jnp.int32)]).reshape(ERP, 128)
    hc, hg = _deg_hist(es, ed)
    es64 = es.reshape(2 * ERP, 64)
    ed64 = ed.reshape(2 * ERP, 64)
    cc_c = _reduce_cc_cells(hc)
    cc_g = _reduce_cc_genes(hg)
    cc_c2 = cc_c.reshape(NCP, 1)
    cc_g2 = cc_g.reshape(NGP, 1)
    u0p = jnp.pad(cell_feat, ((0, NCP - NCELL), (0, 0)))
    g0p = jnp.pad(gene_feat, ((0, NGP - NGENE), (0, 0)))
    a, uh = _prescale_cells(u0p, cc_c2)
    b, ih = _prescale_genes(g0p, cc_g2)
    for _ in range(2):
        csum, gsum = _layer(a, b, es64, ed64)
        uh, a = _update_cells(csum, cc_c2, uh)
        ih, b = _update_genes(gsum, cc_g2, ih)
    gmat = _gmm(uh, ih)
    gf = gmat.reshape(NCP * NGP)
    dpad = DRP * 128 - 2 * E
    zpad = jnp.zeros((dpad,), jnp.int32)
    csrc = jnp.concatenate(
        [pos_src.astype(jnp.int32), neg_src.astype(jnp.int32), zpad]).reshape(DRP, 128)
    cdst = jnp.concatenate(
        [pos_dst.astype(jnp.int32), neg_dst.astype(jnp.int32), zpad]).reshape(DRP, 128)
    (pred2d,) = _decode(gf, csrc, cdst)
    pred = pred2d.reshape(DRP * 128)
    u_hidden = jnp.concatenate([uh[0], uh[1]], axis=1)[:NCELL]
    i_hidden = jnp.concatenate([ih[0], ih[1]], axis=1)[:NGENE]
    return (pred[:E], pred[E:2 * E], u_hidden, i_hidden)
